# baseline (speedup 1.0000x reference)
import jax
import jax.numpy as jnp
from jax import lax
from jax.experimental import pallas as pl
from jax.experimental.pallas import tpu as pltpu
from jax.experimental.pallas import tpu_sc as plsc

RAW = 512
REG = 32
GATHER = 16
L = 16

_NC = 2
_NS = 16
_NW = _NC * _NS

BB = 8


def _sc_attn_body(data_hbm, proto_hbm, feat_hbm, x_v, proto_v, feat_v,
                  xsem, wsem):
    B = data_hbm.shape[0]
    bpw = B // _NW
    sid = lax.axis_index("s")
    wid = sid * _NC + lax.axis_index("c")
    b0 = wid * bpw

    C = data_hbm.shape[2]
    nchunk = C // L

    xcopy = pltpu.async_copy(
        data_hbm.at[pl.ds(b0, bpw), pl.ds(0, GATHER)], x_v, xsem
    )

    pltpu.sync_copy(proto_hbm.at[pl.ds(0, 1)], proto_v)

    zero16 = jnp.zeros((L,), jnp.float32)

    def zrowf(i, carry):
        r = i // 7
        j = 1 + i % 7
        for k in range(nchunk):
            feat_v[r, 0, j, pl.ds(k * L, L)] = zero16
        return carry

    lax.fori_loop(0, bpw * 7, zrowf, 0)

    iota = lax.iota(jnp.int32, L)

    xcopy.wait()

    def batch_body(i, carry):
        b = b0 + i
        sims = zero16
        for j in range(GATHER):
            acc = zero16
            for k in range(nchunk):
                acc = acc + x_v[i, j, pl.ds(k * L, L)] * proto_v[0, pl.ds(k * L, L)]
            sj = jnp.sum(acc) * (1.0 / 16.0)
            sims = jnp.where(iota == j, sj, sims)
        m = jnp.max(sims)
        e = jnp.exp(sims - m)
        attn = e / jnp.sum(e)
        for k in range(nchunk):
            acc = zero16
            for j in range(GATHER):
                acc = acc + attn[j] * x_v[i, j, pl.ds(k * L, L)]
            feat_v[i, 0, 0, pl.ds(k * L, L)] = acc
        pltpu.async_copy(feat_v.at[i], feat_hbm.at[pl.ds(b, 1)], wsem)
        return carry

    lax.fori_loop(0, bpw, batch_body, 0)

    def wdrain(i, carry):
        b = b0 + i
        pltpu.make_async_copy(
            feat_v.at[0], feat_hbm.at[pl.ds(b, 1)], wsem
        ).wait()
        return carry

    lax.fori_loop(0, bpw, wdrain, 0)


def _tc_copy_body(data_ref, out_ref):
    out_ref[:, :RAW, :] = data_ref[...]
    out_ref[:, RAW:, :] = jnp.zeros_like(out_ref[:, RAW:, :])


def _tc_paste_body(base_ref, feat_ref, out_ref):
    del base_ref
    out_ref[...] = feat_ref[...]


@jax.jit
def kernel(data, region_prototypes):
    B, T, C = data.shape
    bpw = B // _NW

    mesh = plsc.VectorSubcoreMesh(core_axis_name="c", subcore_axis_name="s")
    sc_run = pl.kernel(
        _sc_attn_body,
        out_type=jax.ShapeDtypeStruct((B, 8, C), data.dtype),
        mesh=mesh,
        compiler_params=pltpu.CompilerParams(needs_layout_passes=False),
        scratch_types=[
            pltpu.VMEM((bpw, GATHER, C), jnp.float32),
            pltpu.VMEM((1, C), jnp.float32),
            pltpu.VMEM((bpw, 1, 8, C), jnp.float32),
            pltpu.SemaphoreType.DMA,
            pltpu.SemaphoreType.DMA,
        ],
    )
    feat = sc_run(data, region_prototypes)

    base = pl.pallas_call(
        _tc_copy_body,
        out_shape=jax.ShapeDtypeStruct((B, T, C), data.dtype),
        grid=(B // BB,),
        in_specs=[pl.BlockSpec((BB, RAW, C), lambda i: (i, 0, 0))],
        out_specs=pl.BlockSpec((BB, T, C), lambda i: (i, 0, 0)),
    )(data)

    out = pl.pallas_call(
        _tc_paste_body,
        out_shape=jax.ShapeDtypeStruct((B, T, C), data.dtype),
        grid=(B // BB,),
        in_specs=[
            pl.BlockSpec(memory_space=pl.ANY),
            pl.BlockSpec((BB, 8, C), lambda i: (i, 0, 0)),
        ],
        out_specs=pl.BlockSpec((BB, 8, C), lambda i: (i, RAW // 8, 0)),
        input_output_aliases={0: 0},
    )(base, feat)
    return out


# speedup vs baseline: 1.0003x; 1.0003x over previous
import jax
import jax.numpy as jnp
from jax import lax
from jax.experimental import pallas as pl
from jax.experimental.pallas import tpu as pltpu
from jax.experimental.pallas import tpu_sc as plsc

RAW = 512
REG = 32
GATHER = 16
L = 16

_NC = 2
_NS = 16
_NW = _NC * _NS

BB = 8


def _sc_attn_body(data_hbm, proto_hbm, feat_hbm, x_v, proto_v, feat_v,
                  xsem, wsem):
    B = data_hbm.shape[0]
    bpw = B // _NW
    sid = lax.axis_index("s")
    wid = sid * _NC + lax.axis_index("c")
    b0 = wid * bpw

    C = data_hbm.shape[2]
    nchunk = C // L

    xcopy = pltpu.async_copy(
        data_hbm.at[pl.ds(b0, bpw), pl.ds(0, GATHER)], x_v, xsem
    )

    pltpu.sync_copy(proto_hbm.at[pl.ds(0, 1)], proto_v)

    zero16 = jnp.zeros((L,), jnp.float32)

    def zrowf(i, carry):
        r = i // 7
        j = 1 + i % 7
        for k in range(nchunk):
            feat_v[r, 0, j, pl.ds(k * L, L)] = zero16
        return carry

    lax.fori_loop(0, bpw * 7, zrowf, 0)

    iota = lax.iota(jnp.int32, L)

    xcopy.wait()

    def batch_body(i, carry):
        b = b0 + i
        sims = zero16
        for j in range(GATHER):
            acc = zero16
            for k in range(nchunk):
                acc = acc + x_v[i, j, pl.ds(k * L, L)] * proto_v[0, pl.ds(k * L, L)]
            sj = jnp.sum(acc) * (1.0 / 16.0)
            sims = jnp.where(iota == j, sj, sims)
        m = jnp.max(sims)
        e = jnp.exp(sims - m)
        attn = e / jnp.sum(e)
        for k in range(nchunk):
            acc = zero16
            for j in range(GATHER):
                acc = acc + attn[j] * x_v[i, j, pl.ds(k * L, L)]
            feat_v[i, 0, 0, pl.ds(k * L, L)] = acc
        pltpu.async_copy(feat_v.at[i], feat_hbm.at[pl.ds(b, 1)], wsem)
        return carry

    lax.fori_loop(0, bpw, batch_body, 0)

    def wdrain(i, carry):
        b = b0 + i
        pltpu.make_async_copy(
            feat_v.at[0], feat_hbm.at[pl.ds(b, 1)], wsem
        ).wait()
        return carry

    lax.fori_loop(0, bpw, wdrain, 0)


def _tc_copy_body(data_ref, out_ref):
    out_ref[:, :RAW, :] = data_ref[...]
    out_ref[:, RAW:, :] = jnp.zeros_like(out_ref[:, RAW:, :])


def _tc_paste_body(base_ref, feat_ref, out_ref):
    del base_ref
    out_ref[...] = feat_ref[...]


@jax.jit
def kernel(data, region_prototypes):
    B, T, C = data.shape
    bpw = B // _NW

    mesh = plsc.VectorSubcoreMesh(core_axis_name="c", subcore_axis_name="s")
    sc_run = pl.kernel(
        _sc_attn_body,
        out_type=jax.ShapeDtypeStruct((B, 8, C), data.dtype),
        mesh=mesh,
        compiler_params=pltpu.CompilerParams(needs_layout_passes=False),
        scratch_types=[
            pltpu.VMEM((bpw, GATHER, C), jnp.float32),
            pltpu.VMEM((1, C), jnp.float32),
            pltpu.VMEM((bpw, 1, 8, C), jnp.float32),
            pltpu.SemaphoreType.DMA,
            pltpu.SemaphoreType.DMA,
        ],
    )
    base = pl.pallas_call(
        _tc_copy_body,
        out_shape=jax.ShapeDtypeStruct((B, T, C), data.dtype),
        grid=(B // BB,),
        in_specs=[pl.BlockSpec((BB, RAW, C), lambda i: (i, 0, 0))],
        out_specs=pl.BlockSpec((BB, T, C), lambda i: (i, 0, 0)),
    )(data)

    feat = sc_run(data, region_prototypes)

    out = pl.pallas_call(
        _tc_paste_body,
        out_shape=jax.ShapeDtypeStruct((B, T, C), data.dtype),
        grid=(B // BB,),
        in_specs=[
            pl.BlockSpec(memory_space=pl.ANY),
            pl.BlockSpec((BB, 8, C), lambda i: (i, 0, 0)),
        ],
        out_specs=pl.BlockSpec((BB, 8, C), lambda i: (i, RAW // 8, 0)),
        input_output_aliases={0: 0},
    )(base, feat)
    return out


# speedup vs baseline: 1.0587x; 1.0585x over previous
import jax
import jax.numpy as jnp
from jax import lax
from jax.experimental import pallas as pl
from jax.experimental.pallas import tpu as pltpu
from jax.experimental.pallas import tpu_sc as plsc

RAW = 512
REG = 32
GATHER = 16
L = 16

_NC = 2
_NS = 16
_NW = _NC * _NS

BB = 8


def _sc_attn_body(data_hbm, proto_hbm, feat_hbm, x_v, proto_v, feat_v,
                  xsem, wsem):
    B = data_hbm.shape[0]
    bpw = B // _NW
    sid = lax.axis_index("s")
    wid = sid * _NC + lax.axis_index("c")
    b0 = wid * bpw

    C = data_hbm.shape[2]
    nchunk = C // L

    xcopy = pltpu.async_copy(
        data_hbm.at[pl.ds(b0, bpw), pl.ds(0, GATHER)], x_v, xsem
    )

    pltpu.sync_copy(proto_hbm.at[pl.ds(0, 1)], proto_v)

    zero16 = jnp.zeros((L,), jnp.float32)
    iota = lax.iota(jnp.int32, L)

    xcopy.wait()

    def batch_body(i, carry):
        b = b0 + i
        sims = zero16
        for j in range(GATHER):
            acc = zero16
            for k in range(nchunk):
                acc = acc + x_v[i, j, pl.ds(k * L, L)] * proto_v[0, pl.ds(k * L, L)]
            sj = jnp.sum(acc) * (1.0 / 16.0)
            sims = jnp.where(iota == j, sj, sims)
        m = jnp.max(sims)
        e = jnp.exp(sims - m)
        attn = e / jnp.sum(e)
        for k in range(nchunk):
            acc = zero16
            for j in range(GATHER):
                acc = acc + attn[j] * x_v[i, j, pl.ds(k * L, L)]
            feat_v[i, 0, 0, pl.ds(k * L, L)] = acc
        pltpu.async_copy(feat_v.at[i], feat_hbm.at[pl.ds(b, 1)], wsem)
        return carry

    lax.fori_loop(0, bpw, batch_body, 0)

    def wdrain(i, carry):
        b = b0 + i
        pltpu.make_async_copy(
            feat_v.at[0], feat_hbm.at[pl.ds(b, 1)], wsem
        ).wait()
        return carry

    lax.fori_loop(0, bpw, wdrain, 0)


def _tc_assemble_body(data_ref, feat_ref, out_ref):
    out_ref[:, :RAW, :] = data_ref[...]
    out_ref[:, RAW:RAW + 1, :] = feat_ref[:, 0:1, :]
    out_ref[:, RAW + 1:, :] = jnp.zeros_like(out_ref[:, RAW + 1:, :])


@jax.jit
def kernel(data, region_prototypes):
    B, T, C = data.shape
    bpw = B // _NW

    mesh = plsc.VectorSubcoreMesh(core_axis_name="c", subcore_axis_name="s")
    sc_run = pl.kernel(
        _sc_attn_body,
        out_type=jax.ShapeDtypeStruct((B, 8, C), data.dtype),
        mesh=mesh,
        compiler_params=pltpu.CompilerParams(needs_layout_passes=False),
        scratch_types=[
            pltpu.VMEM((bpw, GATHER, C), jnp.float32),
            pltpu.VMEM((1, C), jnp.float32),
            pltpu.VMEM((bpw, 1, 8, C), jnp.float32),
            pltpu.SemaphoreType.DMA,
            pltpu.SemaphoreType.DMA,
        ],
    )
    feat = sc_run(data, region_prototypes)

    out = pl.pallas_call(
        _tc_assemble_body,
        out_shape=jax.ShapeDtypeStruct((B, T, C), data.dtype),
        grid=(B // BB,),
        in_specs=[
            pl.BlockSpec((BB, RAW, C), lambda i: (i, 0, 0)),
            pl.BlockSpec((BB, 8, C), lambda i: (i, 0, 0)),
        ],
        out_specs=pl.BlockSpec((BB, T, C), lambda i: (i, 0, 0)),
    )(data, feat)
    return out


# speedup vs baseline: 1.0588x; 1.0000x over previous
import jax
import jax.numpy as jnp
from jax import lax
from jax.experimental import pallas as pl
from jax.experimental.pallas import tpu as pltpu
from jax.experimental.pallas import tpu_sc as plsc

RAW = 512
REG = 32
GATHER = 16
L = 16

_NC = 2
_NS = 16
_NW = _NC * _NS

BB = 8


def _sc_attn_body(data_hbm, proto_hbm, feat_hbm, x_v, proto_v, feat_v,
                  xsem, wsem):
    B = data_hbm.shape[0]
    bpw = B // _NW
    sid = lax.axis_index("s")
    wid = sid * _NC + lax.axis_index("c")
    b0 = wid * bpw

    C = data_hbm.shape[2]
    nchunk = C // L

    xcopy = pltpu.async_copy(
        data_hbm.at[pl.ds(b0, bpw), pl.ds(0, GATHER)], x_v, xsem
    )

    pltpu.sync_copy(proto_hbm.at[pl.ds(0, 1)], proto_v)

    zero16 = jnp.zeros((L,), jnp.float32)
    iota = lax.iota(jnp.int32, L)

    xcopy.wait()

    def batch_body(i, carry):
        b = b0 + i
        sims = zero16
        for j in range(GATHER):
            acc = zero16
            for k in range(nchunk):
                acc = acc + x_v[i, j, pl.ds(k * L, L)] * proto_v[0, pl.ds(k * L, L)]
            sj = jnp.sum(acc) * (1.0 / 16.0)
            sims = jnp.where(iota == j, sj, sims)
        m = jnp.max(sims)
        e = jnp.exp(sims - m)
        attn = e / jnp.sum(e)
        for k in range(nchunk):
            acc = zero16
            for j in range(GATHER):
                acc = acc + attn[j] * x_v[i, j, pl.ds(k * L, L)]
            feat_v[i, 0, 0, pl.ds(k * L, L)] = acc
        pltpu.async_copy(feat_v.at[i], feat_hbm.at[pl.ds(b, 1)], wsem)
        return carry

    lax.fori_loop(0, bpw, batch_body, 0)

    def wdrain(i, carry):
        b = b0 + i
        pltpu.make_async_copy(
            feat_v.at[0], feat_hbm.at[pl.ds(b, 1)], wsem
        ).wait()
        return carry

    lax.fori_loop(0, bpw, wdrain, 0)


def _tc_assemble_body(data_ref, feat_ref, out_ref):
    out_ref[:, :RAW, :] = data_ref[...]
    out_ref[:, RAW:RAW + 1, :] = feat_ref[:, 0:1, :]
    out_ref[:, RAW + 1:, :] = jnp.zeros_like(out_ref[:, RAW + 1:, :])


@jax.jit
def kernel(data, region_prototypes):
    B, T, C = data.shape
    bpw = B // _NW

    mesh = plsc.VectorSubcoreMesh(core_axis_name="c", subcore_axis_name="s")
    sc_run = pl.kernel(
        _sc_attn_body,
        out_type=jax.ShapeDtypeStruct((B, 8, C), data.dtype),
        mesh=mesh,
        compiler_params=pltpu.CompilerParams(needs_layout_passes=False),
        scratch_types=[
            pltpu.VMEM((bpw, GATHER, C), jnp.float32),
            pltpu.VMEM((1, C), jnp.float32),
            pltpu.VMEM((bpw, 1, 8, C), jnp.float32),
            pltpu.SemaphoreType.DMA,
            pltpu.SemaphoreType.DMA,
        ],
    )
    feat = sc_run(data, region_prototypes)

    out = pl.pallas_call(
        _tc_assemble_body,
        out_shape=jax.ShapeDtypeStruct((B, T, C), data.dtype),
        grid=(B // BB,),
        in_specs=[
            pl.BlockSpec((BB, RAW, C), lambda i: (i, 0, 0)),
            pl.BlockSpec((BB, 8, C), lambda i: (i, 0, 0)),
        ],
        out_specs=pl.BlockSpec((BB, T, C), lambda i: (i, 0, 0)),
        compiler_params=pltpu.CompilerParams(
            dimension_semantics=("parallel",)
        ),
    )(data, feat)
    return out


# speedup vs baseline: 1.0768x; 1.0170x over previous
import jax
import jax.numpy as jnp
from jax import lax
from jax.experimental import pallas as pl
from jax.experimental.pallas import tpu as pltpu
from jax.experimental.pallas import tpu_sc as plsc

RAW = 512
REG = 32
GATHER = 16
L = 16

_NC = 2
_NS = 16
_NW = _NC * _NS

BB = 16


def _sc_attn_body(data_hbm, proto_hbm, feat_hbm, x_v, proto_v, feat_v,
                  xsem, wsem):
    B = data_hbm.shape[0]
    bpw = B // _NW
    sid = lax.axis_index("s")
    wid = sid * _NC + lax.axis_index("c")
    b0 = wid * bpw

    C = data_hbm.shape[2]
    nchunk = C // L

    xcopy = pltpu.async_copy(
        data_hbm.at[pl.ds(b0, bpw), pl.ds(0, GATHER)], x_v, xsem
    )

    pltpu.sync_copy(proto_hbm.at[pl.ds(0, 1)], proto_v)

    zero16 = jnp.zeros((L,), jnp.float32)
    iota = lax.iota(jnp.int32, L)

    xcopy.wait()

    def batch_body(i, carry):
        b = b0 + i
        sims = zero16
        for j in range(GATHER):
            acc = zero16
            for k in range(nchunk):
                acc = acc + x_v[i, j, pl.ds(k * L, L)] * proto_v[0, pl.ds(k * L, L)]
            sj = jnp.sum(acc) * (1.0 / 16.0)
            sims = jnp.where(iota == j, sj, sims)
        m = jnp.max(sims)
        e = jnp.exp(sims - m)
        attn = e / jnp.sum(e)
        for k in range(nchunk):
            acc = zero16
            for j in range(GATHER):
                acc = acc + attn[j] * x_v[i, j, pl.ds(k * L, L)]
            feat_v[i, 0, 0, pl.ds(k * L, L)] = acc
        pltpu.async_copy(feat_v.at[i], feat_hbm.at[pl.ds(b, 1)], wsem)
        return carry

    lax.fori_loop(0, bpw, batch_body, 0)

    def wdrain(i, carry):
        b = b0 + i
        pltpu.make_async_copy(
            feat_v.at[0], feat_hbm.at[pl.ds(b, 1)], wsem
        ).wait()
        return carry

    lax.fori_loop(0, bpw, wdrain, 0)


def _tc_assemble_body(data_ref, feat_ref, out_ref):
    out_ref[:, :RAW, :] = data_ref[...]
    out_ref[:, RAW:RAW + 1, :] = feat_ref[:, 0:1, :]
    out_ref[:, RAW + 1:, :] = jnp.zeros_like(out_ref[:, RAW + 1:, :])


@jax.jit
def kernel(data, region_prototypes):
    B, T, C = data.shape
    bpw = B // _NW

    mesh = plsc.VectorSubcoreMesh(core_axis_name="c", subcore_axis_name="s")
    sc_run = pl.kernel(
        _sc_attn_body,
        out_type=jax.ShapeDtypeStruct((B, 8, C), data.dtype),
        mesh=mesh,
        compiler_params=pltpu.CompilerParams(needs_layout_passes=False),
        scratch_types=[
            pltpu.VMEM((bpw, GATHER, C), jnp.float32),
            pltpu.VMEM((1, C), jnp.float32),
            pltpu.VMEM((bpw, 1, 8, C), jnp.float32),
            pltpu.SemaphoreType.DMA,
            pltpu.SemaphoreType.DMA,
        ],
    )
    feat = sc_run(data, region_prototypes)

    out = pl.pallas_call(
        _tc_assemble_body,
        out_shape=jax.ShapeDtypeStruct((B, T, C), data.dtype),
        grid=(B // BB,),
        in_specs=[
            pl.BlockSpec((BB, RAW, C), lambda i: (i, 0, 0)),
            pl.BlockSpec((BB, 8, C), lambda i: (i, 0, 0)),
        ],
        out_specs=pl.BlockSpec((BB, T, C), lambda i: (i, 0, 0)),
        compiler_params=pltpu.CompilerParams(
            dimension_semantics=("parallel",)
        ),
    )(data, feat)
    return out
